# baseline (speedup 1.0000x reference)
import functools

import jax
import jax.numpy as jnp
from jax import lax
from jax.experimental import pallas as pl
from jax.experimental.pallas import tpu as pltpu
from jax.experimental.pallas import tpu_sc as plsc

N_NODES = 10000
N_EDGES = 160000
D_IN = 256
D_HID = 256
DH = 128
D_OUT = 41
GW = 128

NC = 2
NS = 16
EPT1 = N_EDGES // NS
K1 = 40
C1 = EPT1 // K1
NB1 = 4
EPT2 = N_EDGES // (NC * NS)
K2 = 40
C2 = EPT2 // K2
NB2 = 4
RPT = 640
RPT_LAST = N_NODES - RPT * (NS - 1)
N_PAD = RPT * NS
MB = 1000


def _mm1(x, W1):
    def body(x_ref, w_ref, o_ref):
        o_ref[...] = jnp.dot(x_ref[...], w_ref[...],
                             preferred_element_type=jnp.float32)

    nb = N_NODES // MB
    return pl.pallas_call(
        body,
        grid=(nb, NC),
        in_specs=[pl.BlockSpec((MB, D_IN), lambda i, j: (i, 0)),
                  pl.BlockSpec((D_IN, DH), lambda i, j: (0, j))],
        out_specs=pl.BlockSpec((MB, DH), lambda i, j: (j * nb + i, 0)),
        out_shape=jax.ShapeDtypeStruct((NC * N_NODES, DH), jnp.float32),
    )(x, W1)


def _writeback(acc, out_hbm, c, s):
    @pl.when(s < NS - 1)
    def _():
        pltpu.sync_copy(acc.at[pl.ds(s * RPT, RPT)],
                        out_hbm.at[pl.ds(c * N_NODES + s * RPT, RPT)])

    @pl.when(s == NS - 1)
    def _():
        pltpu.sync_copy(acc.at[pl.ds((NS - 1) * RPT, RPT_LAST)],
                        out_hbm.at[pl.ds(c * N_NODES + (NS - 1) * RPT,
                                         RPT_LAST)])


def _spmm1(h1, col1, row1, z1):
    mesh = plsc.VectorSubcoreMesh(core_axis_name="c", subcore_axis_name="s")

    @functools.partial(
        pl.kernel,
        mesh=mesh,
        out_type=jax.ShapeDtypeStruct((NC * N_NODES, DH), jnp.float32),
        scratch_types=(
            [pltpu.VMEM((K1,), jnp.int32)] * (2 * NB1)
            + [pltpu.VMEM((K1,), jnp.int32)] * (2 * NB1)
            + [pltpu.VMEM((K1, DH), jnp.float32)] * (2 * NB1)
            + [pltpu.VMEM_SHARED((N_PAD, DH), jnp.float32),
               pltpu.SemaphoreType.DMA, pltpu.SemaphoreType.DMA,
               pltpu.SemaphoreType.DMA]),
    )
    def k(h_hbm, col_hbm, row_hbm, z_hbm, out_hbm, *rest):
        colvs = rest[0:2 * NB1]
        rowvs = rest[2 * NB1:4 * NB1]
        gbufs = rest[4 * NB1:6 * NB1]
        acc, semi, semg0, semg1 = rest[6 * NB1:6 * NB1 + 4]
        c = lax.axis_index("c")
        s = lax.axis_index("s")
        pltpu.sync_copy(z_hbm, acc.at[pl.ds(s * RPT, RPT)])
        plsc.subcore_barrier()
        _spmm2_pipeline(h_hbm.at[pl.ds(c * N_NODES, N_NODES)], col_hbm,
                        row_hbm, acc, colvs, rowvs, gbufs,
                        semi, semg0, semg1, s * EPT1, C1, NB1, K1)
        plsc.subcore_barrier()
        _writeback(acc, out_hbm, c, s)

    return k(h1, col1, row1, z1)


def _mm2(s1, W2p):
    def body(a_ref, b_ref, w_ref, o_ref):
        o_ref[...] = (
            jnp.dot(jnp.maximum(a_ref[...], 0.0), w_ref[0:DH, :],
                    preferred_element_type=jnp.float32)
            + jnp.dot(jnp.maximum(b_ref[...], 0.0), w_ref[DH:D_HID, :],
                      preferred_element_type=jnp.float32))

    nb = N_NODES // MB
    return pl.pallas_call(
        body,
        grid=(nb,),
        in_specs=[pl.BlockSpec((MB, DH), lambda i: (i, 0)),
                  pl.BlockSpec((MB, DH), lambda i: (i + nb, 0)),
                  pl.BlockSpec((D_HID, GW), lambda i: (0, 0))],
        out_specs=pl.BlockSpec((MB, GW), lambda i: (i, 0)),
        out_shape=jax.ShapeDtypeStruct((N_NODES, GW), jnp.float32),
    )(s1, s1, W2p)


def _spmm2_pipeline(h_hbm, col_hbm, row_hbm, acc, colvs, rowvs, gbufs,
                    semi, semg0, semg1, base, n_chunks, nb, k):
    n_full = n_chunks // nb
    tail = n_chunks % nb
    semg = [semg0, semg1]

    def fire_idx(par, g):
        for b in range(nb):
            off = base + g * (nb * k) + b * k
            pltpu.make_async_copy(col_hbm.at[pl.ds(off, k)],
                                  colvs[par * nb + b], semi).start()
            pltpu.make_async_copy(row_hbm.at[pl.ds(off, k)],
                                  rowvs[par * nb + b], semi).start()

    def drain_idx(par):
        for b in range(nb):
            pltpu.make_async_copy(col_hbm.at[pl.ds(base, k)],
                                  colvs[par * nb + b], semi).wait()
            pltpu.make_async_copy(row_hbm.at[pl.ds(base, k)],
                                  rowvs[par * nb + b], semi).wait()

    def fire_g(par):
        for b in range(nb):
            pltpu.make_async_copy(h_hbm.at[colvs[par * nb + b]],
                                  gbufs[par * nb + b], semg[par]).start()

    def drain_g(par):
        for b in range(nb):
            pltpu.make_async_copy(h_hbm.at[colvs[par * nb + b]],
                                  gbufs[par * nb + b], semg[par]).wait()

    def scatter(par):
        for b in range(nb):
            pltpu.sync_copy(gbufs[par * nb + b],
                            acc.at[rowvs[par * nb + b]], add=True)

    def half(q, g, g_next):
        drain_idx(q)
        fire_g(q)
        drain_g(1 - q)
        scatter(1 - q)
        fire_idx(1 - q, g_next)

    fire_idx(0, 0)
    drain_idx(0)
    fire_g(0)
    fire_idx(1, 1)

    def outer(i, carry):
        half(1, 2 * i + 1, 2 * i + 2)
        half(0, 2 * i + 2, jnp.minimum(2 * i + 3, n_full - 1))
        return carry

    lax.fori_loop(0, (n_full - 1) // 2, outer, 0)
    if n_full % 2:
        drain_idx(1)
        drain_g(0)
        scatter(0)
    else:
        drain_idx(1)
        fire_g(1)
        drain_g(0)
        scatter(0)
        drain_g(1)
        scatter(1)
    if tail:
        for b in range(tail):
            off = base + n_full * (nb * k) + b * k
            pltpu.make_async_copy(col_hbm.at[pl.ds(off, k)],
                                  colvs[nb + b], semi).start()
            pltpu.make_async_copy(row_hbm.at[pl.ds(off, k)],
                                  rowvs[nb + b], semi).start()
        for b in range(tail):
            pltpu.make_async_copy(col_hbm.at[pl.ds(base, k)],
                                  colvs[nb + b], semi).wait()
            pltpu.make_async_copy(row_hbm.at[pl.ds(base, k)],
                                  rowvs[nb + b], semi).wait()
        for b in range(tail):
            pltpu.make_async_copy(h_hbm.at[colvs[nb + b]],
                                  gbufs[nb + b], semg1).start()
        for b in range(tail):
            pltpu.make_async_copy(h_hbm.at[colvs[nb + b]],
                                  gbufs[nb + b], semg1).wait()
        for b in range(tail):
            pltpu.sync_copy(gbufs[nb + b],
                            acc.at[rowvs[nb + b]], add=True)


def _spmm2(h2, col2, row2, z2):
    mesh = plsc.VectorSubcoreMesh(core_axis_name="c", subcore_axis_name="s")

    @functools.partial(
        pl.kernel,
        mesh=mesh,
        out_type=jax.ShapeDtypeStruct((NC * N_NODES, GW), jnp.float32),
        scratch_types=(
            [pltpu.VMEM((K2,), jnp.int32)] * (2 * NB2)
            + [pltpu.VMEM((K2,), jnp.int32)] * (2 * NB2)
            + [pltpu.VMEM((K2, GW), jnp.float32)] * (2 * NB2)
            + [pltpu.VMEM_SHARED((N_PAD, GW), jnp.float32),
               pltpu.SemaphoreType.DMA, pltpu.SemaphoreType.DMA,
               pltpu.SemaphoreType.DMA]),
    )
    def k(h_hbm, col_hbm, row_hbm, z_hbm, out_hbm, *rest):
        colvs = rest[0:2 * NB2]
        rowvs = rest[2 * NB2:4 * NB2]
        gbufs = rest[4 * NB2:6 * NB2]
        acc, semi, semg0, semg1 = rest[6 * NB2:6 * NB2 + 4]
        c = lax.axis_index("c")
        s = lax.axis_index("s")
        t = c * NS + s
        pltpu.sync_copy(z_hbm, acc.at[pl.ds(s * RPT, RPT)])
        plsc.subcore_barrier()
        _spmm2_pipeline(h_hbm, col_hbm, row_hbm, acc, colvs, rowvs, gbufs,
                        semi, semg0, semg1, t * EPT2, C2, NB2, K2)
        plsc.subcore_barrier()
        _writeback(acc, out_hbm, c, s)

    return k(h2, col2, row2, z2)


def _final_add(p):
    def body(a_ref, b_ref, o_ref):
        o_ref[...] = a_ref[:, :D_OUT] + b_ref[:, :D_OUT]

    nb = N_NODES // MB
    return pl.pallas_call(
        body,
        grid=(nb,),
        in_specs=[pl.BlockSpec((MB, GW), lambda i: (i, 0)),
                  pl.BlockSpec((MB, GW), lambda i: (i + nb, 0))],
        out_specs=pl.BlockSpec((MB, D_OUT), lambda i: (i, 0)),
        out_shape=jax.ShapeDtypeStruct((N_NODES, D_OUT), jnp.float32),
    )(p, p)


def kernel(edge_index, x, W1, W2):
    row = edge_index[0]
    col = edge_index[1]

    z1 = jnp.zeros((RPT, DH), jnp.float32)
    W2p = jnp.pad(W2, ((0, 0), (0, GW - D_OUT)))

    h1 = _mm1(x, W1)
    s1 = _spmm1(h1, col, row, z1)
    h2 = _mm2(s1, W2p)
    p = _spmm2(h2, col, row, z1)
    return _final_add(p)


# speedup vs baseline: 1.0170x; 1.0170x over previous
import functools

import jax
import jax.numpy as jnp
from jax import lax
from jax.experimental import pallas as pl
from jax.experimental.pallas import tpu as pltpu
from jax.experimental.pallas import tpu_sc as plsc

N_NODES = 10000
N_EDGES = 160000
D_IN = 256
D_HID = 256
DH = 128
D_OUT = 41
GW = 128

NC = 2
NS = 16
EPT1 = N_EDGES // NS
K1 = 80
C1 = EPT1 // K1
NB1 = 2
EPT2 = N_EDGES // (NC * NS)
K2 = 40
C2 = EPT2 // K2
NB2 = 4
RPT = 640
RPT_LAST = N_NODES - RPT * (NS - 1)
N_PAD = RPT * NS
MB = 1000


def _mm1(x, W1):
    def body(x_ref, w_ref, o_ref):
        o_ref[...] = jnp.dot(x_ref[...], w_ref[...],
                             preferred_element_type=jnp.float32)

    nb = N_NODES // MB
    return pl.pallas_call(
        body,
        grid=(nb, NC),
        in_specs=[pl.BlockSpec((MB, D_IN), lambda i, j: (i, 0)),
                  pl.BlockSpec((D_IN, DH), lambda i, j: (0, j))],
        out_specs=pl.BlockSpec((MB, DH), lambda i, j: (j * nb + i, 0)),
        out_shape=jax.ShapeDtypeStruct((NC * N_NODES, DH), jnp.float32),
    )(x, W1)


def _writeback(acc, out_hbm, c, s):
    @pl.when(s < NS - 1)
    def _():
        pltpu.sync_copy(acc.at[pl.ds(s * RPT, RPT)],
                        out_hbm.at[pl.ds(c * N_NODES + s * RPT, RPT)])

    @pl.when(s == NS - 1)
    def _():
        pltpu.sync_copy(acc.at[pl.ds((NS - 1) * RPT, RPT_LAST)],
                        out_hbm.at[pl.ds(c * N_NODES + (NS - 1) * RPT,
                                         RPT_LAST)])


def _spmm1(h1, col1, row1, z1):
    mesh = plsc.VectorSubcoreMesh(core_axis_name="c", subcore_axis_name="s")

    @functools.partial(
        pl.kernel,
        mesh=mesh,
        out_type=jax.ShapeDtypeStruct((NC * N_NODES, DH), jnp.float32),
        scratch_types=(
            [pltpu.VMEM((K1,), jnp.int32)] * (2 * NB1)
            + [pltpu.VMEM((K1,), jnp.int32)] * (2 * NB1)
            + [pltpu.VMEM((K1, DH), jnp.float32)] * (2 * NB1)
            + [pltpu.VMEM_SHARED((N_PAD, DH), jnp.float32),
               pltpu.SemaphoreType.DMA, pltpu.SemaphoreType.DMA,
               pltpu.SemaphoreType.DMA]),
    )
    def k(h_hbm, col_hbm, row_hbm, z_hbm, out_hbm, *rest):
        colvs = rest[0:2 * NB1]
        rowvs = rest[2 * NB1:4 * NB1]
        gbufs = rest[4 * NB1:6 * NB1]
        acc, semi, semg0, semg1 = rest[6 * NB1:6 * NB1 + 4]
        c = lax.axis_index("c")
        s = lax.axis_index("s")
        pltpu.sync_copy(z_hbm, acc.at[pl.ds(s * RPT, RPT)])
        plsc.subcore_barrier()
        _spmm2_pipeline(h_hbm.at[pl.ds(c * N_NODES, N_NODES)], col_hbm,
                        row_hbm, acc, colvs, rowvs, gbufs,
                        semi, semg0, semg1, s * EPT1, C1, NB1, K1)
        plsc.subcore_barrier()
        _writeback(acc, out_hbm, c, s)

    return k(h1, col1, row1, z1)


def _mm2(s1, W2p):
    def body(a_ref, b_ref, w_ref, o_ref):
        o_ref[...] = (
            jnp.dot(jnp.maximum(a_ref[...], 0.0), w_ref[0:DH, :],
                    preferred_element_type=jnp.float32)
            + jnp.dot(jnp.maximum(b_ref[...], 0.0), w_ref[DH:D_HID, :],
                      preferred_element_type=jnp.float32))

    nb = N_NODES // MB
    return pl.pallas_call(
        body,
        grid=(nb,),
        in_specs=[pl.BlockSpec((MB, DH), lambda i: (i, 0)),
                  pl.BlockSpec((MB, DH), lambda i: (i + nb, 0)),
                  pl.BlockSpec((D_HID, GW), lambda i: (0, 0))],
        out_specs=pl.BlockSpec((MB, GW), lambda i: (i, 0)),
        out_shape=jax.ShapeDtypeStruct((N_NODES, GW), jnp.float32),
    )(s1, s1, W2p)


def _spmm2_pipeline(h_hbm, col_hbm, row_hbm, acc, colvs, rowvs, gbufs,
                    semi, semg0, semg1, base, n_chunks, nb, k):
    n_full = n_chunks // nb
    tail = n_chunks % nb
    semg = [semg0, semg1]

    def fire_idx(par, g):
        for b in range(nb):
            off = base + g * (nb * k) + b * k
            pltpu.make_async_copy(col_hbm.at[pl.ds(off, k)],
                                  colvs[par * nb + b], semi).start()
            pltpu.make_async_copy(row_hbm.at[pl.ds(off, k)],
                                  rowvs[par * nb + b], semi).start()

    def drain_idx(par):
        for b in range(nb):
            pltpu.make_async_copy(col_hbm.at[pl.ds(base, k)],
                                  colvs[par * nb + b], semi).wait()
            pltpu.make_async_copy(row_hbm.at[pl.ds(base, k)],
                                  rowvs[par * nb + b], semi).wait()

    def fire_g(par):
        for b in range(nb):
            pltpu.make_async_copy(h_hbm.at[colvs[par * nb + b]],
                                  gbufs[par * nb + b], semg[par]).start()

    def drain_g(par):
        for b in range(nb):
            pltpu.make_async_copy(h_hbm.at[colvs[par * nb + b]],
                                  gbufs[par * nb + b], semg[par]).wait()

    def scatter(par):
        for b in range(nb):
            pltpu.sync_copy(gbufs[par * nb + b],
                            acc.at[rowvs[par * nb + b]], add=True)

    def half(q, g, g_next):
        drain_idx(q)
        fire_g(q)
        drain_g(1 - q)
        scatter(1 - q)
        fire_idx(1 - q, g_next)

    fire_idx(0, 0)
    drain_idx(0)
    fire_g(0)
    fire_idx(1, 1)

    def outer(i, carry):
        half(1, 2 * i + 1, 2 * i + 2)
        half(0, 2 * i + 2, jnp.minimum(2 * i + 3, n_full - 1))
        return carry

    lax.fori_loop(0, (n_full - 1) // 2, outer, 0)
    if n_full % 2:
        drain_idx(1)
        drain_g(0)
        scatter(0)
    else:
        drain_idx(1)
        fire_g(1)
        drain_g(0)
        scatter(0)
        drain_g(1)
        scatter(1)
    if tail:
        for b in range(tail):
            off = base + n_full * (nb * k) + b * k
            pltpu.make_async_copy(col_hbm.at[pl.ds(off, k)],
                                  colvs[nb + b], semi).start()
            pltpu.make_async_copy(row_hbm.at[pl.ds(off, k)],
                                  rowvs[nb + b], semi).start()
        for b in range(tail):
            pltpu.make_async_copy(col_hbm.at[pl.ds(base, k)],
                                  colvs[nb + b], semi).wait()
            pltpu.make_async_copy(row_hbm.at[pl.ds(base, k)],
                                  rowvs[nb + b], semi).wait()
        for b in range(tail):
            pltpu.make_async_copy(h_hbm.at[colvs[nb + b]],
                                  gbufs[nb + b], semg1).start()
        for b in range(tail):
            pltpu.make_async_copy(h_hbm.at[colvs[nb + b]],
                                  gbufs[nb + b], semg1).wait()
        for b in range(tail):
            pltpu.sync_copy(gbufs[nb + b],
                            acc.at[rowvs[nb + b]], add=True)


def _spmm2(h2, col2, row2, z2):
    mesh = plsc.VectorSubcoreMesh(core_axis_name="c", subcore_axis_name="s")

    @functools.partial(
        pl.kernel,
        mesh=mesh,
        out_type=jax.ShapeDtypeStruct((NC * N_NODES, GW), jnp.float32),
        scratch_types=(
            [pltpu.VMEM((K2,), jnp.int32)] * (2 * NB2)
            + [pltpu.VMEM((K2,), jnp.int32)] * (2 * NB2)
            + [pltpu.VMEM((K2, GW), jnp.float32)] * (2 * NB2)
            + [pltpu.VMEM_SHARED((N_PAD, GW), jnp.float32),
               pltpu.SemaphoreType.DMA, pltpu.SemaphoreType.DMA,
               pltpu.SemaphoreType.DMA]),
    )
    def k(h_hbm, col_hbm, row_hbm, z_hbm, out_hbm, *rest):
        colvs = rest[0:2 * NB2]
        rowvs = rest[2 * NB2:4 * NB2]
        gbufs = rest[4 * NB2:6 * NB2]
        acc, semi, semg0, semg1 = rest[6 * NB2:6 * NB2 + 4]
        c = lax.axis_index("c")
        s = lax.axis_index("s")
        t = c * NS + s
        pltpu.sync_copy(z_hbm, acc.at[pl.ds(s * RPT, RPT)])
        plsc.subcore_barrier()
        _spmm2_pipeline(h_hbm, col_hbm, row_hbm, acc, colvs, rowvs, gbufs,
                        semi, semg0, semg1, t * EPT2, C2, NB2, K2)
        plsc.subcore_barrier()
        _writeback(acc, out_hbm, c, s)

    return k(h2, col2, row2, z2)


def _final_add(p):
    def body(a_ref, b_ref, o_ref):
        o_ref[...] = a_ref[:, :D_OUT] + b_ref[:, :D_OUT]

    nb = N_NODES // MB
    return pl.pallas_call(
        body,
        grid=(nb,),
        in_specs=[pl.BlockSpec((MB, GW), lambda i: (i, 0)),
                  pl.BlockSpec((MB, GW), lambda i: (i + nb, 0))],
        out_specs=pl.BlockSpec((MB, D_OUT), lambda i: (i, 0)),
        out_shape=jax.ShapeDtypeStruct((N_NODES, D_OUT), jnp.float32),
    )(p, p)


def kernel(edge_index, x, W1, W2):
    row = edge_index[0]
    col = edge_index[1]

    z1 = jnp.zeros((RPT, DH), jnp.float32)
    W2p = jnp.pad(W2, ((0, 0), (0, GW - D_OUT)))

    h1 = _mm1(x, W1)
    s1 = _spmm1(h1, col, row, z1)
    h2 = _mm2(s1, W2p)
    p = _spmm2(h2, col, row, z1)
    return _final_add(p)
